# baseline (speedup 1.0000x reference)
import functools

import jax
import jax.numpy as jnp
from jax import lax
from jax.experimental import pallas as pl
from jax.experimental.pallas import tpu as pltpu
from jax.experimental.pallas import tpu_sc as plsc

_NUM_WORKERS = 32


def _chunk_size(bpw: int) -> int:
    for ch in range(min(bpw, 1024), 0, -8):
        if bpw % ch == 0:
            return ch
    return bpw


@functools.partial(jax.jit, static_argnames=())
def kernel(input, table):
    flat = input.reshape(-1).astype(jnp.int32)
    b_total = flat.shape[0]
    d = table.shape[1]
    bpw = b_total // _NUM_WORKERS
    ch = _chunk_size(bpw)
    n_chunks = bpw // ch

    mesh = plsc.VectorSubcoreMesh(core_axis_name="c", subcore_axis_name="s")

    @functools.partial(
        pl.kernel,
        mesh=mesh,
        compiler_params=pltpu.CompilerParams(use_tc_tiling_on_sc=False),
        out_type=jax.ShapeDtypeStruct((b_total, d), jnp.float32),
        scratch_types=[
            pltpu.VMEM((bpw,), jnp.int32),
            pltpu.VMEM((ch, d), jnp.float32),
            pltpu.VMEM((ch, d), jnp.float32),
            pltpu.SemaphoreType.DMA,
            pltpu.SemaphoreType.DMA,
            pltpu.SemaphoreType.DMA,
            pltpu.SemaphoreType.DMA,
        ],
    )
    def gather_kernel(idx_hbm, table_hbm, out_hbm, idx_v, rows0, rows1,
                      g0, g1, w0, w1):
        wid = lax.axis_index("s") * 2 + lax.axis_index("c")
        base = wid * bpw
        pltpu.sync_copy(idx_hbm.at[pl.ds(base, bpw)], idx_v)

        rows = (rows0, rows1)
        gsem = (g0, g1)
        wsem = (w0, w1)

        def gather(c, b):
            return pltpu.async_copy(
                table_hbm.at[idx_v.at[pl.ds(c * ch, ch)]], rows[b], gsem[b])

        def write(c, b):
            return pltpu.async_copy(
                rows[b], out_hbm.at[pl.ds(base + c * ch, ch)], wsem[b])

        g = [None, None]
        w = [None, None]
        g[0] = gather(0, 0)
        for c in range(n_chunks):
            b = c & 1
            nb = 1 - b
            g[b].wait()
            if c + 1 < n_chunks:
                if w[nb] is not None:
                    w[nb].wait()
                g[nb] = gather(c + 1, nb)
            w[b] = write(c, b)
        if n_chunks >= 2:
            w[(n_chunks - 2) & 1].wait()
        w[(n_chunks - 1) & 1].wait()

    return gather_kernel(flat, table)


# speedup vs baseline: 1.0117x; 1.0117x over previous
import functools

import jax
import jax.numpy as jnp
from jax import lax
from jax.experimental import pallas as pl
from jax.experimental.pallas import tpu as pltpu
from jax.experimental.pallas import tpu_sc as plsc

_NUM_WORKERS = 32


def _chunk_size(bpw: int) -> int:
    for ch in range(min(bpw, 400), 0, -8):
        if bpw % ch == 0:
            return ch
    return bpw


@functools.partial(jax.jit, static_argnames=())
def kernel(input, table):
    flat = input.reshape(-1).astype(jnp.int32)
    b_total = flat.shape[0]
    d = table.shape[1]
    bpw = b_total // _NUM_WORKERS
    ch = _chunk_size(bpw)
    n_chunks = bpw // ch
    nbuf = min(4, n_chunks)

    mesh = plsc.VectorSubcoreMesh(core_axis_name="c", subcore_axis_name="s")

    @functools.partial(
        pl.kernel,
        mesh=mesh,
        compiler_params=pltpu.CompilerParams(use_tc_tiling_on_sc=False),
        out_type=jax.ShapeDtypeStruct((b_total, d), jnp.float32),
        scratch_types=(
            [pltpu.VMEM((bpw,), jnp.int32)]
            + [pltpu.VMEM((ch, d), jnp.float32) for _ in range(nbuf)]
            + [pltpu.SemaphoreType.DMA for _ in range(2 * nbuf)]
        ),
    )
    def gather_kernel(idx_hbm, table_hbm, out_hbm, idx_v, *bufs_and_sems):
        rows = bufs_and_sems[:nbuf]
        gsem = bufs_and_sems[nbuf:2 * nbuf]
        wsem = bufs_and_sems[2 * nbuf:3 * nbuf]

        wid = lax.axis_index("s") * 2 + lax.axis_index("c")
        base = wid * bpw
        pltpu.sync_copy(idx_hbm.at[pl.ds(base, bpw)], idx_v)

        def gather(c, b):
            return pltpu.async_copy(
                table_hbm.at[idx_v.at[pl.ds(c * ch, ch)]], rows[b], gsem[b])

        def write(c, b):
            return pltpu.async_copy(
                rows[b], out_hbm.at[pl.ds(base + c * ch, ch)], wsem[b])

        g = [gather(k, k) for k in range(nbuf)]
        w = [None] * nbuf
        for c in range(n_chunks):
            b = c % nbuf
            g[b].wait()
            w[b] = write(c, b)
            nc = c + nbuf
            if nc < n_chunks:
                w[b].wait()
                g[b] = gather(nc, b)
        for k in range(max(0, n_chunks - nbuf), n_chunks):
            w[k % nbuf].wait()

    return gather_kernel(flat, table)
